# Initial kernel scaffold; baseline (speedup 1.0000x reference)
import functools

import jax
import jax.numpy as jnp
from jax import lax
from jax.experimental import pallas as pl
from jax.experimental.pallas import tpu as pltpu
from jax.experimental.pallas import tpu_sc as plsc

DIM = 64
COND_DIM = 256

_INFO = plsc.get_sparse_core_info()
_NC, _NS = _INFO.num_cores, _INFO.num_subcores
_NW = _NC * _NS

_IDX_CHUNK = 128


def _make_sc_gather(batch, dim):
    assert batch % (8 * _NW) == 0
    b_per_w = batch // _NW
    assert b_per_w % _IDX_CHUNK == 0
    n_chunks = b_per_w // _IDX_CHUNK
    mesh = plsc.VectorSubcoreMesh(core_axis_name="c", subcore_axis_name="s")

    @functools.partial(
        pl.kernel,
        mesh=mesh,
        out_type=jax.ShapeDtypeStruct((batch, dim), jnp.float32),
        scratch_types=[
            pltpu.VMEM((b_per_w,), jnp.int32),
            pltpu.VMEM((b_per_w, dim), jnp.float32),
            pltpu.SemaphoreType.DMA,
        ],
    )
    def gather(table_hbm, idx_hbm, out_hbm, idx_v, rows_v, sem):
        wid = lax.axis_index("s") * _NC + lax.axis_index("c")
        base = wid * b_per_w
        pltpu.sync_copy(idx_hbm.at[pl.ds(base, b_per_w)], idx_v)
        copies = [
            pltpu.async_copy(
                table_hbm.at[idx_v.at[pl.ds(j * _IDX_CHUNK, _IDX_CHUNK)]],
                rows_v.at[pl.ds(j * _IDX_CHUNK, _IDX_CHUNK)],
                sem,
            )
            for j in range(n_chunks)
        ]
        for c in copies:
            c.wait()
        pltpu.sync_copy(rows_v, out_hbm.at[pl.ds(base, b_per_w)])

    return gather


def _mlp_body(x_ref, w_ref, b_ref, o_ref):
    y = jnp.dot(x_ref[...], w_ref[...], preferred_element_type=jnp.float32)
    y = y + b_ref[...]
    o_ref[...] = y * jax.nn.sigmoid(y)


def _make_tc_mlp(batch, dim, cond_dim, blk):
    assert batch % blk == 0
    return pl.pallas_call(
        _mlp_body,
        grid=(batch // blk,),
        in_specs=[
            pl.BlockSpec((blk, dim), lambda i: (i, 0)),
            pl.BlockSpec((dim, cond_dim), lambda i: (0, 0)),
            pl.BlockSpec((1, cond_dim), lambda i: (0, 0)),
        ],
        out_specs=pl.BlockSpec((blk, cond_dim), lambda i: (i, 0)),
        out_shape=jax.ShapeDtypeStruct((batch, cond_dim), jnp.float32),
    )


def kernel(condition, cond_embed, W, b):
    idx = condition.reshape(-1).astype(jnp.int32)
    batch = idx.shape[0]
    gathered = _make_sc_gather(batch, DIM)(cond_embed, idx)
    mlp = _make_tc_mlp(batch, DIM, COND_DIM, blk=2048)
    return mlp(gathered, W, b.reshape(1, COND_DIM))


# speedup vs baseline: 1.0459x; 1.0459x over previous
import functools

import jax
import jax.numpy as jnp
from jax import lax
from jax.experimental import pallas as pl
from jax.experimental.pallas import tpu as pltpu
from jax.experimental.pallas import tpu_sc as plsc

DIM = 64
COND_DIM = 256
TILE_ROWS = 8

_CHUNK = 32
_LANES = 16


def _sc_workers():
    try:
        info = plsc.get_sparse_core_info()
        return info.num_cores, info.num_subcores
    except Exception:
        return 2, 16


def _make_sc_gather(batch, dim):
    nc, ns = _sc_workers()
    nw = nc * ns
    assert batch % (8 * nw) == 0
    b_per_w = batch // nw
    assert b_per_w % _CHUNK == 0
    n_chunks = b_per_w // _CHUNK
    mesh = plsc.VectorSubcoreMesh(core_axis_name="c", subcore_axis_name="s")

    @functools.partial(
        pl.kernel,
        mesh=mesh,
        out_type=jax.ShapeDtypeStruct((batch, dim), jnp.float32),
        scratch_types=[
            pltpu.VMEM((b_per_w,), jnp.int32),
            pltpu.VMEM((_CHUNK, TILE_ROWS, dim), jnp.float32),
            pltpu.VMEM((b_per_w, dim), jnp.float32),
            pltpu.SemaphoreType.DMA,
        ],
    )
    def gather(table_hbm, idx_hbm, out_hbm, idx_v, tiles_v, rows_v, sem):
        wid = lax.axis_index("s") * nc + lax.axis_index("c")
        base = wid * b_per_w
        pltpu.sync_copy(idx_hbm.at[pl.ds(base, b_per_w)], idx_v)

        def chunk_body(c, carry):
            off = c * _CHUNK
            tvecs, svecs = [], []
            for g in range(_CHUNK // _LANES):
                v = idx_v[pl.ds(off + g * _LANES, _LANES)]
                tvecs.append(lax.shift_right_logical(v, 3))
                svecs.append(v & 7)
            copies = []
            for k in range(_CHUNK):
                t = tvecs[k // _LANES][k % _LANES]
                copies.append(
                    pltpu.async_copy(table_hbm.at[t], tiles_v.at[k], sem)
                )
            for cp in copies:
                cp.wait()
            for k in range(_CHUNK):
                sub = svecs[k // _LANES][k % _LANES]
                for q in range(dim // _LANES):
                    sl = pl.ds(q * _LANES, _LANES)
                    rows_v[off + k, sl] = tiles_v[k, sub, sl]
            return carry

        lax.fori_loop(0, n_chunks, chunk_body, 0, unroll=False)
        pltpu.sync_copy(rows_v, out_hbm.at[pl.ds(base, b_per_w)])

    return gather


def _mlp_body(x_ref, w_ref, b_ref, o_ref):
    y = jnp.dot(x_ref[...], w_ref[...], preferred_element_type=jnp.float32)
    y = y + b_ref[...]
    o_ref[...] = y * jax.nn.sigmoid(y)


def _make_tc_mlp(batch, dim, cond_dim, blk):
    assert batch % blk == 0
    return pl.pallas_call(
        _mlp_body,
        grid=(batch // blk,),
        in_specs=[
            pl.BlockSpec((blk, dim), lambda i: (i, 0)),
            pl.BlockSpec((dim, cond_dim), lambda i: (0, 0)),
            pl.BlockSpec((1, cond_dim), lambda i: (0, 0)),
        ],
        out_specs=pl.BlockSpec((blk, cond_dim), lambda i: (i, 0)),
        out_shape=jax.ShapeDtypeStruct((batch, cond_dim), jnp.float32),
    )


def kernel(condition, cond_embed, W, b):
    idx = condition.reshape(-1).astype(jnp.int32)
    batch = idx.shape[0]
    table3 = cond_embed.reshape(-1, TILE_ROWS, DIM)
    gathered = _make_sc_gather(batch, DIM)(table3, idx)
    mlp = _make_tc_mlp(batch, DIM, COND_DIM, blk=2048)
    return mlp(gathered, W, b.reshape(1, COND_DIM))
